# baseline (speedup 1.0000x reference)
import functools

import jax
import jax.numpy as jnp
from jax import lax
from jax.experimental import pallas as pl
from jax.experimental.pallas import tpu as pltpu
from jax.experimental.pallas import tpu_sc as plsc

_N, _C, _H, _W = 4, 96, 384, 384
_HW = _H * _W
_P = _N * _HW
_NW = 32
_PW = _P // _NW
_B = 128
_NCHUNK = _PW // _B
_OROWS = _N * _C * _HW // _B


def _sc_body(table, gx_h, gy_h, out_h,
             gxv, gyv,
             idxA, wA, rowsA0, rowsA1, rowsA2, rowsA3, outA, oidxA,
             idxB, wB, rowsB0, rowsB1, rowsB2, rowsB3, outB, oidxB,
             semA, semB, semOutA, semOutB):
    cid = lax.axis_index("c")
    sid = lax.axis_index("s")
    wid = sid * 2 + cid
    nbatch = wid // (_HW // _PW)
    nbase = nbatch * _HW
    base = wid * _PW
    lanes = lax.iota(jnp.int32, 16)

    def stage(g, idx4, w4, rows, sem):
        cb = base + g * _B
        pltpu.sync_copy(gx_h.at[pl.ds(cb, _B)], gxv)
        pltpu.sync_copy(gy_h.at[pl.ds(cb, _B)], gyv)
        for j in range(_B // 16):
            s = pl.ds(j * 16, 16)
            x = (gxv[s] + 1.0) * (_W * 0.5) - 0.5
            y = (gyv[s] + 1.0) * (_H * 0.5) - 0.5
            xt = x.astype(jnp.int32)
            xtf = xt.astype(jnp.float32)
            x0f = jnp.where(x < xtf, xtf - 1.0, xtf)
            x0 = x0f.astype(jnp.int32)
            yt = y.astype(jnp.int32)
            ytf = yt.astype(jnp.float32)
            y0f = jnp.where(y < ytf, ytf - 1.0, ytf)
            y0 = y0f.astype(jnp.int32)
            x1 = x0 + 1
            y1 = y0 + 1
            wx1 = x - x0f
            wx0 = 1.0 - wx1
            wy1 = y - y0f
            wy0 = 1.0 - wy1
            vx0 = (x0 >= 0) & (x0 <= _W - 1)
            vx1 = (x1 >= 0) & (x1 <= _W - 1)
            vy0 = (y0 >= 0) & (y0 <= _H - 1)
            vy1 = (y1 >= 0) & (y1 <= _H - 1)
            zero = jnp.zeros((16,), jnp.float32)
            xc0 = jnp.clip(x0, 0, _W - 1)
            xc1 = jnp.clip(x1, 0, _W - 1)
            yc0 = jnp.clip(y0, 0, _H - 1)
            yc1 = jnp.clip(y1, 0, _H - 1)
            r0 = nbase + yc0 * _W
            r1 = nbase + yc1 * _W
            idx4[0, s] = r0 + xc0
            idx4[1, s] = r0 + xc1
            idx4[2, s] = r1 + xc0
            idx4[3, s] = r1 + xc1
            w4[0, s] = jnp.where(vx0 & vy0, wx0 * wy0, zero)
            w4[1, s] = jnp.where(vx1 & vy0, wx1 * wy0, zero)
            w4[2, s] = jnp.where(vx0 & vy1, wx0 * wy1, zero)
            w4[3, s] = jnp.where(vx1 & vy1, wx1 * wy1, zero)
        pltpu.async_copy(table.at[idx4.at[0]], rows[0], sem)
        pltpu.async_copy(table.at[idx4.at[1]], rows[1], sem)
        pltpu.async_copy(table.at[idx4.at[2]], rows[2], sem)
        pltpu.async_copy(table.at[idx4.at[3]], rows[3], sem)

    def finish(g, idx4, w4, rows, outbuf, oidx, sem, sem_out, first):
        cb = base + g * _B
        for k in range(4):
            pltpu.make_async_copy(table.at[idx4.at[k]], rows[k], sem).wait()
        @pl.when(jnp.logical_not(first))
        def _drain_prev():
            pltpu.make_async_copy(outbuf, out_h.at[oidx], sem_out).wait()

        def acc_body(j, acc_carry):
            off = pl.multiple_of(j * 16, 16)
            s16 = pl.ds(off, 16)
            w00v = w4[0, s16]
            w01v = w4[1, s16]
            w10v = w4[2, s16]
            w11v = w4[3, s16]
            for l in range(16):
                p = off + l
                psplat = jnp.full((16,), p, jnp.int32)
                w00f = jnp.full((16,), w00v[l])
                w01f = jnp.full((16,), w01v[l])
                w10f = jnp.full((16,), w10v[l])
                w11f = jnp.full((16,), w11v[l])
                w00 = plsc.pack(w00f, w00f, format=plsc.PackFormat.INTERLEAVED)
                w01 = plsc.pack(w01f, w01f, format=plsc.PackFormat.INTERLEAVED)
                w10 = plsc.pack(w10f, w10f, format=plsc.PackFormat.INTERLEAVED)
                w11 = plsc.pack(w11f, w11f, format=plsc.PackFormat.INTERLEAVED)
                for cg in range(_C // 32):
                    s = pl.ds(cg * 32, 32)
                    o = (w00 * rows[0][p, s] + w01 * rows[1][p, s]
                         + w10 * rows[2][p, s] + w11 * rows[3][p, s])
                    oe, oo = plsc.unpack(o, format=plsc.PackFormat.INTERLEAVED)
                    ce = cg * 32 + 2 * lanes
                    plsc.store_scatter(outbuf, [ce, psplat], oe)
                    plsc.store_scatter(outbuf, [ce + 1, psplat], oo)
            return acc_carry

        lax.fori_loop(0, _B // 16, acc_body, 0)
        orow0 = (nbatch * _C) * (_HW // _B) + (cb - nbase) // _B
        for j in range(_C // 16):
            s = pl.ds(j * 16, 16)
            oidx[s] = orow0 + (j * 16 + lanes) * (_HW // _B)
        pltpu.async_copy(outbuf, out_h.at[oidx], sem_out)

    rowsA = (rowsA0, rowsA1, rowsA2, rowsA3)
    rowsB = (rowsB0, rowsB1, rowsB2, rowsB3)

    stage(0, idxA, wA, rowsA, semA)
    f_true = jnp.bool_(True)
    f_false = jnp.bool_(False)

    def pair_body(i, carry):
        g0 = i * 2
        first = i == 0
        stage(g0 + 1, idxB, wB, rowsB, semB)
        finish(g0, idxA, wA, rowsA, outA, oidxA, semA, semOutA, first)
        stage(g0 + 2, idxA, wA, rowsA, semA)
        finish(g0 + 1, idxB, wB, rowsB, outB, oidxB, semB, semOutB, first)
        return carry

    lax.fori_loop(0, (_NCHUNK - 2) // 2, pair_body, 0)

    g_last = _NCHUNK - 2
    stage(g_last + 1, idxB, wB, rowsB, semB)
    finish(g_last, idxA, wA, rowsA, outA, oidxA, semA, semOutA, f_false)
    finish(g_last + 1, idxB, wB, rowsB, outB, oidxB, semB, semOutB, f_false)
    pltpu.make_async_copy(outA, out_h.at[oidxA], semOutA).wait()
    pltpu.make_async_copy(outB, out_h.at[oidxB], semOutB).wait()


_sc_call = functools.partial(
    pl.kernel,
    out_type=jax.ShapeDtypeStruct((_OROWS, _B), jnp.float32),
    mesh=plsc.VectorSubcoreMesh(core_axis_name="c", subcore_axis_name="s"),
    compiler_params=pltpu.CompilerParams(
        use_tc_tiling_on_sc=False, needs_layout_passes=False),
    scratch_types=[
        pltpu.VMEM((_B,), jnp.float32),
        pltpu.VMEM((_B,), jnp.float32),
        pltpu.VMEM((4, _B), jnp.int32),
        pltpu.VMEM((4, _B), jnp.float32),
        pltpu.VMEM((_B, _C), jnp.bfloat16),
        pltpu.VMEM((_B, _C), jnp.bfloat16),
        pltpu.VMEM((_B, _C), jnp.bfloat16),
        pltpu.VMEM((_B, _C), jnp.bfloat16),
        pltpu.VMEM((_C, _B), jnp.float32),
        pltpu.VMEM((_C,), jnp.int32),
        pltpu.VMEM((4, _B), jnp.int32),
        pltpu.VMEM((4, _B), jnp.float32),
        pltpu.VMEM((_B, _C), jnp.bfloat16),
        pltpu.VMEM((_B, _C), jnp.bfloat16),
        pltpu.VMEM((_B, _C), jnp.bfloat16),
        pltpu.VMEM((_B, _C), jnp.bfloat16),
        pltpu.VMEM((_C, _B), jnp.float32),
        pltpu.VMEM((_C,), jnp.int32),
        pltpu.SemaphoreType.DMA,
        pltpu.SemaphoreType.DMA,
        pltpu.SemaphoreType.DMA,
        pltpu.SemaphoreType.DMA,
    ],
)(_sc_body)


@jax.jit
def kernel(input, grid):
    n, c, h, w = input.shape
    assert (n, c, h, w) == (_N, _C, _H, _W)
    table = input.transpose(0, 2, 3, 1).reshape(_P, _C).astype(jnp.bfloat16)
    gx = grid[..., 0].reshape(_P)
    gy = grid[..., 1].reshape(_P)
    out = _sc_call(table, gx, gy)
    return out.reshape(_N, _C, _H, _W)


# speedup vs baseline: 1.0444x; 1.0444x over previous
import functools

import jax
import jax.numpy as jnp
from jax import lax
from jax.experimental import pallas as pl
from jax.experimental.pallas import tpu as pltpu
from jax.experimental.pallas import tpu_sc as plsc

_N, _C, _H, _W = 4, 96, 384, 384
_P = _N * _H * _W
_NW = 32
_PW = _P // _NW
_B = 128
_NCHUNK = _PW // _B


def _sc_body(table, gx_h, gy_h, out_h,
             gxv, gyv,
             idxA, wA, rowsA0, rowsA1, rowsA2, rowsA3,
             idxB, wB, rowsB0, rowsB1, rowsB2, rowsB3,
             outbuf, semA, semB):
    cid = lax.axis_index("c")
    sid = lax.axis_index("s")
    wid = sid * 2 + cid
    nbase = (wid // (_H * _W // _PW)) * (_H * _W)
    base = wid * _PW

    def stage(g, idx4, w4, rows, sem):
        cb = base + g * _B
        pltpu.sync_copy(gx_h.at[pl.ds(cb, _B)], gxv)
        pltpu.sync_copy(gy_h.at[pl.ds(cb, _B)], gyv)
        for j in range(_B // 16):
            s = pl.ds(j * 16, 16)
            x = (gxv[s] + 1.0) * (_W * 0.5) - 0.5
            y = (gyv[s] + 1.0) * (_H * 0.5) - 0.5
            xt = x.astype(jnp.int32)
            xtf = xt.astype(jnp.float32)
            x0f = jnp.where(x < xtf, xtf - 1.0, xtf)
            x0 = x0f.astype(jnp.int32)
            yt = y.astype(jnp.int32)
            ytf = yt.astype(jnp.float32)
            y0f = jnp.where(y < ytf, ytf - 1.0, ytf)
            y0 = y0f.astype(jnp.int32)
            x1 = x0 + 1
            y1 = y0 + 1
            wx1 = x - x0f
            wx0 = 1.0 - wx1
            wy1 = y - y0f
            wy0 = 1.0 - wy1
            vx0 = (x0 >= 0) & (x0 <= _W - 1)
            vx1 = (x1 >= 0) & (x1 <= _W - 1)
            vy0 = (y0 >= 0) & (y0 <= _H - 1)
            vy1 = (y1 >= 0) & (y1 <= _H - 1)
            zero = jnp.zeros((16,), jnp.float32)
            xc0 = jnp.clip(x0, 0, _W - 1)
            xc1 = jnp.clip(x1, 0, _W - 1)
            yc0 = jnp.clip(y0, 0, _H - 1)
            yc1 = jnp.clip(y1, 0, _H - 1)
            r0 = nbase + yc0 * _W
            r1 = nbase + yc1 * _W
            idx4[0, s] = r0 + xc0
            idx4[1, s] = r0 + xc1
            idx4[2, s] = r1 + xc0
            idx4[3, s] = r1 + xc1
            w4[0, s] = jnp.where(vx0 & vy0, wx0 * wy0, zero)
            w4[1, s] = jnp.where(vx1 & vy0, wx1 * wy0, zero)
            w4[2, s] = jnp.where(vx0 & vy1, wx0 * wy1, zero)
            w4[3, s] = jnp.where(vx1 & vy1, wx1 * wy1, zero)
        pltpu.async_copy(table.at[idx4.at[0]], rows[0], sem)
        pltpu.async_copy(table.at[idx4.at[1]], rows[1], sem)
        pltpu.async_copy(table.at[idx4.at[2]], rows[2], sem)
        pltpu.async_copy(table.at[idx4.at[3]], rows[3], sem)

    def finish(g, idx4, w4, rows, sem):
        cb = base + g * _B
        for k in range(4):
            pltpu.make_async_copy(table.at[idx4.at[k]], rows[k], sem).wait()

        def acc_body(j, acc_carry):
            off = pl.multiple_of(j * 16, 16)
            s16 = pl.ds(off, 16)
            w00v = w4[0, s16]
            w01v = w4[1, s16]
            w10v = w4[2, s16]
            w11v = w4[3, s16]
            for l in range(16):
                p = off + l
                w00f = jnp.full((16,), w00v[l])
                w01f = jnp.full((16,), w01v[l])
                w10f = jnp.full((16,), w10v[l])
                w11f = jnp.full((16,), w11v[l])
                w00 = plsc.pack(w00f, w00f, format=plsc.PackFormat.INTERLEAVED)
                w01 = plsc.pack(w01f, w01f, format=plsc.PackFormat.INTERLEAVED)
                w10 = plsc.pack(w10f, w10f, format=plsc.PackFormat.INTERLEAVED)
                w11 = plsc.pack(w11f, w11f, format=plsc.PackFormat.INTERLEAVED)
                for cg in range(_C // 32):
                    s = pl.ds(cg * 32, 32)
                    outbuf[p, s] = (
                        w00 * rows[0][p, s] + w01 * rows[1][p, s]
                        + w10 * rows[2][p, s] + w11 * rows[3][p, s])
            return acc_carry

        lax.fori_loop(0, _B // 16, acc_body, 0)
        pltpu.sync_copy(outbuf, out_h.at[pl.ds(cb, _B)])

    rowsA = (rowsA0, rowsA1, rowsA2, rowsA3)
    rowsB = (rowsB0, rowsB1, rowsB2, rowsB3)

    stage(0, idxA, wA, rowsA, semA)

    def pair_body(i, carry):
        g0 = i * 2
        stage(g0 + 1, idxB, wB, rowsB, semB)
        finish(g0, idxA, wA, rowsA, semA)
        stage(g0 + 2, idxA, wA, rowsA, semA)
        finish(g0 + 1, idxB, wB, rowsB, semB)
        return carry

    lax.fori_loop(0, (_NCHUNK - 2) // 2, pair_body, 0)

    g_last = _NCHUNK - 2
    stage(g_last + 1, idxB, wB, rowsB, semB)
    finish(g_last, idxA, wA, rowsA, semA)
    finish(g_last + 1, idxB, wB, rowsB, semB)


_sc_call = functools.partial(
    pl.kernel,
    out_type=jax.ShapeDtypeStruct((_P, _C), jnp.bfloat16),
    mesh=plsc.VectorSubcoreMesh(core_axis_name="c", subcore_axis_name="s"),
    compiler_params=pltpu.CompilerParams(
        use_tc_tiling_on_sc=False, needs_layout_passes=False),
    scratch_types=[
        pltpu.VMEM((_B,), jnp.float32),
        pltpu.VMEM((_B,), jnp.float32),
        pltpu.VMEM((4, _B), jnp.int32),
        pltpu.VMEM((4, _B), jnp.float32),
        pltpu.VMEM((_B, _C), jnp.bfloat16),
        pltpu.VMEM((_B, _C), jnp.bfloat16),
        pltpu.VMEM((_B, _C), jnp.bfloat16),
        pltpu.VMEM((_B, _C), jnp.bfloat16),
        pltpu.VMEM((4, _B), jnp.int32),
        pltpu.VMEM((4, _B), jnp.float32),
        pltpu.VMEM((_B, _C), jnp.bfloat16),
        pltpu.VMEM((_B, _C), jnp.bfloat16),
        pltpu.VMEM((_B, _C), jnp.bfloat16),
        pltpu.VMEM((_B, _C), jnp.bfloat16),
        pltpu.VMEM((_B, _C), jnp.bfloat16),
        pltpu.SemaphoreType.DMA,
        pltpu.SemaphoreType.DMA,
    ],
)(_sc_body)


_HW = _H * _W
_BLK = 2048


def _eye_bf16():
    ii = lax.broadcasted_iota(jnp.int32, (_C, _C), 0)
    jj = lax.broadcasted_iota(jnp.int32, (_C, _C), 1)
    return (ii == jj).astype(jnp.bfloat16)


def _to_table_body(x_ref, o_ref):
    x = x_ref[0].astype(jnp.bfloat16)
    o_ref[0] = lax.dot_general(
        x, _eye_bf16(), (((0,), (0,)), ((), ())),
        preferred_element_type=jnp.float32).astype(jnp.bfloat16)


_to_table = pl.pallas_call(
    _to_table_body,
    grid=(_N, _HW // _BLK),
    in_specs=[pl.BlockSpec((1, _C, _BLK), lambda n, i: (n, 0, i))],
    out_specs=pl.BlockSpec((1, _BLK, _C), lambda n, i: (n, i, 0)),
    out_shape=jax.ShapeDtypeStruct((_N, _HW, _C), jnp.bfloat16),
)


def _from_nhwc_body(y_ref, o_ref):
    o_ref[0] = lax.dot_general(
        _eye_bf16(), y_ref[0], (((1,), (1,)), ((), ())),
        preferred_element_type=jnp.float32)


_from_nhwc = pl.pallas_call(
    _from_nhwc_body,
    grid=(_N, _HW // _BLK),
    in_specs=[pl.BlockSpec((1, _BLK, _C), lambda n, i: (n, i, 0))],
    out_specs=pl.BlockSpec((1, _C, _BLK), lambda n, i: (n, 0, i)),
    out_shape=jax.ShapeDtypeStruct((_N, _C, _HW), jnp.float32),
)


@jax.jit
def kernel(input, grid):
    n, c, h, w = input.shape
    assert (n, c, h, w) == (_N, _C, _H, _W)
    table = _to_table(input.reshape(_N, _C, _HW)).reshape(_P, _C)
    gx = grid[..., 0].reshape(_P)
    gy = grid[..., 1].reshape(_P)
    out = _sc_call(table, gx, gy)
    return _from_nhwc(out.reshape(_N, _HW, _C)).reshape(_N, _C, _H, _W)


# speedup vs baseline: 1.3546x; 1.2970x over previous
import functools

import jax
import jax.numpy as jnp
from jax import lax
from jax.experimental import pallas as pl
from jax.experimental.pallas import tpu as pltpu
from jax.experimental.pallas import tpu_sc as plsc

_N, _C, _H, _W = 4, 96, 384, 384
_P = _N * _H * _W
_NW = 32
_PW = _P // _NW
_B = 128
_NCHUNK = _PW // _B


def _sc_body(table, gx_h, gy_h, out_h,
             gxv, gyv,
             idxA, wA, rowsA0, rowsA1, rowsA2, rowsA3,
             idxB, wB, rowsB0, rowsB1, rowsB2, rowsB3,
             outbuf, semA, semB):
    cid = lax.axis_index("c")
    sid = lax.axis_index("s")
    wid = sid * 2 + cid
    nbase = (wid // (_H * _W // _PW)) * (_H * _W)
    base = wid * _PW

    def stage(g, idx4, w4, rows, sem):
        cb = base + g * _B
        pltpu.sync_copy(gx_h.at[pl.ds(cb, _B)], gxv)
        pltpu.sync_copy(gy_h.at[pl.ds(cb, _B)], gyv)
        for j in range(_B // 16):
            s = pl.ds(j * 16, 16)
            x = (gxv[s] + 1.0) * (_W * 0.5) - 0.5
            y = (gyv[s] + 1.0) * (_H * 0.5) - 0.5
            xt = x.astype(jnp.int32)
            xtf = xt.astype(jnp.float32)
            x0f = jnp.where(x < xtf, xtf - 1.0, xtf)
            x0 = x0f.astype(jnp.int32)
            yt = y.astype(jnp.int32)
            ytf = yt.astype(jnp.float32)
            y0f = jnp.where(y < ytf, ytf - 1.0, ytf)
            y0 = y0f.astype(jnp.int32)
            x1 = x0 + 1
            y1 = y0 + 1
            wx1 = x - x0f
            wx0 = 1.0 - wx1
            wy1 = y - y0f
            wy0 = 1.0 - wy1
            vx0 = (x0 >= 0) & (x0 <= _W - 1)
            vx1 = (x1 >= 0) & (x1 <= _W - 1)
            vy0 = (y0 >= 0) & (y0 <= _H - 1)
            vy1 = (y1 >= 0) & (y1 <= _H - 1)
            zero = jnp.zeros((16,), jnp.float32)
            xc0 = jnp.clip(x0, 0, _W - 1)
            xc1 = jnp.clip(x1, 0, _W - 1)
            yc0 = jnp.clip(y0, 0, _H - 1)
            yc1 = jnp.clip(y1, 0, _H - 1)
            r0 = nbase + yc0 * _W
            r1 = nbase + yc1 * _W
            idx4[0, s] = r0 + xc0
            idx4[1, s] = r0 + xc1
            idx4[2, s] = r1 + xc0
            idx4[3, s] = r1 + xc1
            w4[0, s] = jnp.where(vx0 & vy0, wx0 * wy0, zero)
            w4[1, s] = jnp.where(vx1 & vy0, wx1 * wy0, zero)
            w4[2, s] = jnp.where(vx0 & vy1, wx0 * wy1, zero)
            w4[3, s] = jnp.where(vx1 & vy1, wx1 * wy1, zero)
        pltpu.async_copy(table.at[idx4.at[0]], rows[0], sem)
        pltpu.async_copy(table.at[idx4.at[1]], rows[1], sem)
        pltpu.async_copy(table.at[idx4.at[2]], rows[2], sem)
        pltpu.async_copy(table.at[idx4.at[3]], rows[3], sem)

    def finish(g, idx4, w4, rows, sem):
        cb = base + g * _B
        for k in range(4):
            pltpu.make_async_copy(table.at[idx4.at[k]], rows[k], sem).wait()

        def acc_body(j, acc_carry):
            off = pl.multiple_of(j * 16, 16)
            s16 = pl.ds(off, 16)
            w00v = w4[0, s16]
            w01v = w4[1, s16]
            w10v = w4[2, s16]
            w11v = w4[3, s16]
            for l in range(16):
                p = off + l
                w00 = jnp.full((16,), w00v[l])
                w01 = jnp.full((16,), w01v[l])
                w10 = jnp.full((16,), w10v[l])
                w11 = jnp.full((16,), w11v[l])
                for cg in range(_C // 16):
                    s = pl.ds(cg * 16, 16)
                    outbuf[p, s] = (
                        w00 * rows[0][p, s] + w01 * rows[1][p, s]
                        + w10 * rows[2][p, s] + w11 * rows[3][p, s])
            return acc_carry

        lax.fori_loop(0, _B // 16, acc_body, 0)
        pltpu.sync_copy(outbuf, out_h.at[pl.ds(cb, _B)])

    rowsA = (rowsA0, rowsA1, rowsA2, rowsA3)
    rowsB = (rowsB0, rowsB1, rowsB2, rowsB3)

    stage(0, idxA, wA, rowsA, semA)

    def pair_body(i, carry):
        g0 = i * 2
        stage(g0 + 1, idxB, wB, rowsB, semB)
        finish(g0, idxA, wA, rowsA, semA)
        stage(g0 + 2, idxA, wA, rowsA, semA)
        finish(g0 + 1, idxB, wB, rowsB, semB)
        return carry

    lax.fori_loop(0, (_NCHUNK - 2) // 2, pair_body, 0)

    g_last = _NCHUNK - 2
    stage(g_last + 1, idxB, wB, rowsB, semB)
    finish(g_last, idxA, wA, rowsA, semA)
    finish(g_last + 1, idxB, wB, rowsB, semB)


_sc_call = functools.partial(
    pl.kernel,
    out_type=jax.ShapeDtypeStruct((_P, _C), jnp.float32),
    mesh=plsc.VectorSubcoreMesh(core_axis_name="c", subcore_axis_name="s"),
    compiler_params=pltpu.CompilerParams(use_tc_tiling_on_sc=False),
    scratch_types=[
        pltpu.VMEM((_B,), jnp.float32),
        pltpu.VMEM((_B,), jnp.float32),
        pltpu.VMEM((4, _B), jnp.int32),
        pltpu.VMEM((4, _B), jnp.float32),
        pltpu.VMEM((_B, _C), jnp.float32),
        pltpu.VMEM((_B, _C), jnp.float32),
        pltpu.VMEM((_B, _C), jnp.float32),
        pltpu.VMEM((_B, _C), jnp.float32),
        pltpu.VMEM((4, _B), jnp.int32),
        pltpu.VMEM((4, _B), jnp.float32),
        pltpu.VMEM((_B, _C), jnp.float32),
        pltpu.VMEM((_B, _C), jnp.float32),
        pltpu.VMEM((_B, _C), jnp.float32),
        pltpu.VMEM((_B, _C), jnp.float32),
        pltpu.VMEM((_B, _C), jnp.float32),
        pltpu.SemaphoreType.DMA,
        pltpu.SemaphoreType.DMA,
    ],
)(_sc_body)


@jax.jit
def kernel(input, grid):
    n, c, h, w = input.shape
    assert (n, c, h, w) == (_N, _C, _H, _W)
    table = input.transpose(0, 2, 3, 1).reshape(_P, _C)
    gx = grid[..., 0].reshape(_P)
    gy = grid[..., 1].reshape(_P)
    out = _sc_call(table, gx, gy)
    return out.reshape(_N, _H, _W, _C).transpose(0, 3, 1, 2)
